# Initial kernel scaffold; baseline (speedup 1.0000x reference)
import jax
import jax.numpy as jnp
from jax.experimental import pallas as pl


def kernel(x, table, Wp, Wt0, bt0, Wg0, bg0, Wt1, bt1, Wg1, bg1):
    raise NotImplementedError("write your pallas kernel here")


# speedup vs baseline: 8.3863x; 8.3863x over previous
import functools

import jax
import jax.numpy as jnp
from jax import lax
from jax.experimental import pallas as pl
from jax.experimental.pallas import tpu as pltpu
from jax.experimental.pallas import tpu_sc as plsc

VOCAB, EDIM, HID = 100000, 64, 128
B, L = 4096, 200
TOKENS = B * L

VBLK = 2000

NC, NS = 2, 16
NW = NC * NS
ROWS_PER_W = TOKENS // NW
CHUNK = 128
NCHUNK = ROWS_PER_W // CHUNK
TOTAL_CHUNKS = TOKENS // CHUNK


def _sigmoid(x):
    return 1.0 / (1.0 + jnp.exp(-x))


def _mlp_body(table_ref, wp_ref, wt0_ref, bt0_ref, wg0_ref, bg0_ref,
              wt1_ref, bt1_ref, wg1_ref, bg1_ref, out_ref):
    h = jnp.dot(table_ref[...], wp_ref[...], preferred_element_type=jnp.float32)
    for wt, bt, wg, bg in ((wt0_ref, bt0_ref, wg0_ref, bg0_ref),
                           (wt1_ref, bt1_ref, wg1_ref, bg1_ref)):
        g = _sigmoid(jnp.dot(h, wg[...], preferred_element_type=jnp.float32) + bg[...])
        t = jnp.maximum(jnp.dot(h, wt[...], preferred_element_type=jnp.float32) + bt[...], 0.0)
        h = g * t + (1.0 - g) * h
    out_ref[...] = h


def _precompute_table(table, Wp, Wt0, bt0, Wg0, bg0, Wt1, bt1, Wg1, bg1):
    wp_spec = pl.BlockSpec((EDIM, HID), lambda i: (0, 0))
    w_spec = pl.BlockSpec((HID, HID), lambda i: (0, 0))
    b_spec = pl.BlockSpec((1, HID), lambda i: (0, 0))
    return pl.pallas_call(
        _mlp_body,
        grid=(VOCAB // VBLK,),
        in_specs=[pl.BlockSpec((VBLK, EDIM), lambda i: (i, 0)),
                  wp_spec, w_spec, b_spec, w_spec, b_spec,
                  w_spec, b_spec, w_spec, b_spec],
        out_specs=pl.BlockSpec((VBLK, HID), lambda i: (i, 0)),
        out_shape=jax.ShapeDtypeStruct((VOCAB, HID), jnp.float32),
    )(table, Wp, Wt0, bt0.reshape(1, HID), Wg0, bg0.reshape(1, HID),
      Wt1, bt1.reshape(1, HID), Wg1, bg1.reshape(1, HID))


@functools.lru_cache(maxsize=1)
def _make_sc_gather():
    @functools.partial(
        pl.kernel,
        mesh=plsc.VectorSubcoreMesh(core_axis_name="c", subcore_axis_name="s"),
        out_type=jax.ShapeDtypeStruct((TOTAL_CHUNKS, CHUNK, HID), jnp.float32),
        scratch_types=[
            pltpu.VMEM((NCHUNK, CHUNK), jnp.int32),
            pltpu.VMEM((CHUNK, HID), jnp.float32),
            pltpu.SemaphoreType.DMA,
        ],
    )
    def _sc_gather(table2_hbm, idx_hbm, out_hbm, idx_v, rows_v, sem):
        wid = lax.axis_index("s") * NC + lax.axis_index("c")
        chunk0 = wid * NCHUNK
        pltpu.sync_copy(idx_hbm.at[pl.ds(chunk0, NCHUNK)], idx_v)

        def body(j, carry):
            pltpu.async_copy(table2_hbm.at[idx_v.at[j]], rows_v, sem).wait()
            pltpu.sync_copy(rows_v, out_hbm.at[chunk0 + j])
            return carry

        lax.fori_loop(0, NCHUNK, body, 0)

    return _sc_gather


def kernel(x, table, Wp, Wt0, bt0, Wg0, bg0, Wt1, bt1, Wg1, bg1):
    table2 = _precompute_table(table, Wp, Wt0, bt0, Wg0, bg0, Wt1, bt1, Wg1, bg1)
    idx = x.reshape(TOTAL_CHUNKS, CHUNK)
    out = _make_sc_gather()(table2, idx)
    return out.reshape(B, L, HID)


# speedup vs baseline: 11.2502x; 1.3415x over previous
import functools

import jax
import jax.numpy as jnp
from jax import lax
from jax.experimental import pallas as pl
from jax.experimental.pallas import tpu as pltpu
from jax.experimental.pallas import tpu_sc as plsc

VOCAB, EDIM, HID = 100000, 64, 128
B, L = 4096, 200
TOKENS = B * L

VBLK = 2000

NC, NS = 2, 16
NW = NC * NS
ROWS_PER_W = TOKENS // NW
CHUNK = 128
NCHUNK = ROWS_PER_W // CHUNK
TOTAL_CHUNKS = TOKENS // CHUNK


def _sigmoid(x):
    return 1.0 / (1.0 + jnp.exp(-x))


def _mlp_body(table_ref, wp_ref, wt0_ref, bt0_ref, wg0_ref, bg0_ref,
              wt1_ref, bt1_ref, wg1_ref, bg1_ref, out_ref):
    h = jnp.dot(table_ref[...], wp_ref[...], preferred_element_type=jnp.float32)
    for wt, bt, wg, bg in ((wt0_ref, bt0_ref, wg0_ref, bg0_ref),
                           (wt1_ref, bt1_ref, wg1_ref, bg1_ref)):
        g = _sigmoid(jnp.dot(h, wg[...], preferred_element_type=jnp.float32) + bg[...])
        t = jnp.maximum(jnp.dot(h, wt[...], preferred_element_type=jnp.float32) + bt[...], 0.0)
        h = g * t + (1.0 - g) * h
    out_ref[...] = h


def _precompute_table(table, Wp, Wt0, bt0, Wg0, bg0, Wt1, bt1, Wg1, bg1):
    wp_spec = pl.BlockSpec((EDIM, HID), lambda i: (0, 0))
    w_spec = pl.BlockSpec((HID, HID), lambda i: (0, 0))
    b_spec = pl.BlockSpec((1, HID), lambda i: (0, 0))
    return pl.pallas_call(
        _mlp_body,
        grid=(VOCAB // VBLK,),
        in_specs=[pl.BlockSpec((VBLK, EDIM), lambda i: (i, 0)),
                  wp_spec, w_spec, b_spec, w_spec, b_spec,
                  w_spec, b_spec, w_spec, b_spec],
        out_specs=pl.BlockSpec((VBLK, HID), lambda i: (i, 0)),
        out_shape=jax.ShapeDtypeStruct((VOCAB, HID), jnp.float32),
    )(table, Wp, Wt0, bt0.reshape(1, HID), Wg0, bg0.reshape(1, HID),
      Wt1, bt1.reshape(1, HID), Wg1, bg1.reshape(1, HID))


@functools.lru_cache(maxsize=1)
def _make_sc_gather():
    @functools.partial(
        pl.kernel,
        mesh=plsc.VectorSubcoreMesh(core_axis_name="c", subcore_axis_name="s"),
        out_type=jax.ShapeDtypeStruct((TOTAL_CHUNKS, CHUNK, HID), jnp.float32),
        scratch_types=[
            pltpu.VMEM((NCHUNK, CHUNK), jnp.int32),
            pltpu.VMEM((CHUNK, HID), jnp.float32),
            pltpu.VMEM((CHUNK, HID), jnp.float32),
            pltpu.SemaphoreType.DMA,
            pltpu.SemaphoreType.DMA,
        ],
    )
    def _sc_gather(table2_hbm, idx_hbm, out_hbm, idx_v, rows0, rows1, sem0, sem1):
        wid = lax.axis_index("s") * NC + lax.axis_index("c")
        chunk0 = wid * NCHUNK
        pltpu.sync_copy(idx_hbm.at[pl.ds(chunk0, NCHUNK)], idx_v)
        pltpu.async_copy(table2_hbm.at[idx_v.at[0]], rows0, sem0)

        def body(jj, carry):
            j0 = jj * 2
            pltpu.async_copy(table2_hbm.at[idx_v.at[j0 + 1]], rows1, sem1)
            pltpu.make_async_copy(table2_hbm.at[idx_v.at[j0]], rows0, sem0).wait()
            pltpu.sync_copy(rows0, out_hbm.at[chunk0 + j0])

            @pl.when(j0 + 2 < NCHUNK)
            def _():
                pltpu.async_copy(table2_hbm.at[idx_v.at[j0 + 2]], rows0, sem0)

            pltpu.make_async_copy(table2_hbm.at[idx_v.at[j0 + 1]], rows1, sem1).wait()
            pltpu.sync_copy(rows1, out_hbm.at[chunk0 + j0 + 1])
            return carry

        lax.fori_loop(0, NCHUNK // 2, body, 0)

    return _sc_gather


def kernel(x, table, Wp, Wt0, bt0, Wg0, bg0, Wt1, bt1, Wg1, bg1):
    table2 = _precompute_table(table, Wp, Wt0, bt0, Wg0, bg0, Wt1, bt1, Wg1, bg1)
    idx = x.reshape(TOTAL_CHUNKS, CHUNK)
    out = _make_sc_gather()(table2, idx)
    return out.reshape(B, L, HID)
